# baseline (speedup 1.0000x reference)
import functools

import jax
import jax.numpy as jnp
from jax import lax
from jax.experimental import pallas as pl
from jax.experimental.pallas import tpu as pltpu
from jax.experimental.pallas import tpu_sc as plsc

EMBED_DIM = 32
BATCH = 16384

NUM_CORES = 2
NUM_SUBCORES = 16
NUM_WORKERS = NUM_CORES * NUM_SUBCORES
B_PER_W = BATCH // NUM_WORKERS

_mesh = plsc.VectorSubcoreMesh(core_axis_name="c", subcore_axis_name="s")


@functools.partial(
    pl.kernel,
    mesh=_mesh,
    out_type=jax.ShapeDtypeStruct((4, 128, 8, 128), jnp.float32),
    scratch_types=[
        pltpu.VMEM((B_PER_W,), jnp.int32),
        pltpu.VMEM((B_PER_W, EMBED_DIM), jnp.float32),
        pltpu.VMEM((4, 4, 8, 128), jnp.float32),
        pltpu.SemaphoreType.DMA,
    ],
    compiler_params=pltpu.CompilerParams(
        use_tc_tiling_on_sc=False, needs_layout_passes=False
    ),
)
def _embed_gather(idx_hbm, table_hbm, out_hbm, idx_v, rows_v, outbuf, sem):
    wid = lax.axis_index("s") * NUM_CORES + lax.axis_index("c")
    base = wid * B_PER_W
    pltpu.sync_copy(idx_hbm.at[pl.ds(base, B_PER_W)], idx_v)
    pltpu.async_copy(table_hbm.at[idx_v], rows_v, sem).wait()

    iota = lax.iota(jnp.int32, 16)

    @plsc.parallel_loop(0, B_PER_W // 16, step=1, unroll=8)
    def _transpose(g):
        rowsel = iota + g * 16
        jtvec = jnp.broadcast_to(g // 8, (16,))
        bvec = iota + (g % 8) * 16
        for j0 in range(EMBED_DIM):
            csel = (iota + j0) & (EMBED_DIM - 1)
            v = plsc.load_gather(rows_v, [rowsel, csel])
            plsc.store_scatter(outbuf, [csel >> 3, jtvec, csel & 7, bvec], v)

    jt0 = base // 128
    for ib in range(4):
        pltpu.sync_copy(outbuf.at[ib], out_hbm.at[ib, pl.ds(jt0, 4)])


def kernel(user_id, table):
    out4 = _embed_gather(user_id, table)
    return out4.transpose(1, 3, 0, 2).reshape(BATCH, EMBED_DIM)


# speedup vs baseline: 1.2595x; 1.2595x over previous
import functools

import jax
import jax.numpy as jnp
from jax import lax
from jax.experimental import pallas as pl
from jax.experimental.pallas import tpu as pltpu
from jax.experimental.pallas import tpu_sc as plsc

EMBED_DIM = 32
BATCH = 16384

NUM_CORES = 2
NUM_SUBCORES = 16
NUM_WORKERS = NUM_CORES * NUM_SUBCORES
B_PER_W = BATCH // NUM_WORKERS

_mesh = plsc.VectorSubcoreMesh(core_axis_name="c", subcore_axis_name="s")


@functools.partial(
    pl.kernel,
    mesh=_mesh,
    out_type=jax.ShapeDtypeStruct((4, 128, 8, 128), jnp.float32),
    scratch_types=[
        pltpu.VMEM((B_PER_W,), jnp.int32),
        pltpu.VMEM((B_PER_W, EMBED_DIM), jnp.float32),
        pltpu.VMEM((4, 4, 8, 128), jnp.float32),
        pltpu.SemaphoreType.DMA,
        pltpu.SemaphoreType.DMA,
        pltpu.SemaphoreType.DMA,
        pltpu.SemaphoreType.DMA,
        pltpu.SemaphoreType.DMA,
    ],
    compiler_params=pltpu.CompilerParams(
        use_tc_tiling_on_sc=False, needs_layout_passes=False
    ),
)
def _embed_gather(
    idx_hbm, table_hbm, out_hbm, idx_v, rows_v, outbuf, g0, g1, g2, g3, osem
):
    wid = lax.axis_index("s") * NUM_CORES + lax.axis_index("c")
    base = wid * B_PER_W
    jt0 = base // 128
    pltpu.sync_copy(idx_hbm.at[pl.ds(base, B_PER_W)], idx_v)

    gsems = [g0, g1, g2, g3]
    gathers = [
        pltpu.async_copy(
            table_hbm.at[idx_v.at[pl.ds(c * 128, 128)]],
            rows_v.at[pl.ds(c * 128, 128)],
            gsems[c],
        )
        for c in range(4)
    ]

    iota = lax.iota(jnp.int32, 16)
    cols = [jnp.full((16,), j, jnp.int32) for j in range(EMBED_DIM)]
    outs = []
    for c in range(4):
        gathers[c].wait()

        @plsc.parallel_loop(c * 8, c * 8 + 8, step=1, unroll=8)
        def _transpose(g):
            rowsel = iota + g * 16
            jt = g // 8
            boff = (g % 8) * 16
            for j in range(EMBED_DIM):
                v = plsc.load_gather(rows_v, [rowsel, cols[j]])
                outbuf[j // 8, jt, j % 8, pl.ds(boff, 16)] = v

        for ib in range(4):
            outs.append(
                pltpu.async_copy(outbuf.at[ib, c], out_hbm.at[ib, jt0 + c], osem)
            )
    for o in outs:
        o.wait()


def kernel(user_id, table):
    out4 = _embed_gather(user_id, table)
    return out4.transpose(1, 3, 0, 2).reshape(BATCH, EMBED_DIM)


# speedup vs baseline: 1.6241x; 1.2894x over previous
import functools

import jax
import jax.numpy as jnp
from jax import lax
from jax.experimental import pallas as pl
from jax.experimental.pallas import tpu as pltpu
from jax.experimental.pallas import tpu_sc as plsc

EMBED_DIM = 32
BATCH = 16384

NUM_CORES = 2
NUM_SUBCORES = 16
NUM_WORKERS = NUM_CORES * NUM_SUBCORES
B_PER_W = BATCH // NUM_WORKERS

_mesh = plsc.VectorSubcoreMesh(core_axis_name="c", subcore_axis_name="s")


@functools.partial(
    pl.kernel,
    mesh=_mesh,
    out_type=jax.ShapeDtypeStruct((4, 128, 8, 128), jnp.float32),
    scratch_types=[
        pltpu.VMEM((B_PER_W,), jnp.int32),
        pltpu.VMEM((B_PER_W, EMBED_DIM + 1), jnp.float32),
        pltpu.VMEM((4, 4, 8, 128), jnp.float32),
        pltpu.SemaphoreType.DMA,
    ],
    compiler_params=pltpu.CompilerParams(
        use_tc_tiling_on_sc=False, needs_layout_passes=False
    ),
)
def _embed_gather(idx_hbm, table_hbm, out_hbm, idx_v, rows_v, outbuf, sem):
    wid = lax.axis_index("s") * NUM_CORES + lax.axis_index("c")
    base = wid * B_PER_W
    pltpu.sync_copy(idx_hbm.at[pl.ds(base, B_PER_W)], idx_v)
    pltpu.async_copy(table_hbm.at[idx_v], rows_v, sem).wait()

    iota = lax.iota(jnp.int32, 16)
    cols = [jnp.full((16,), j, jnp.int32) for j in range(EMBED_DIM)]

    @plsc.parallel_loop(0, B_PER_W // 16, step=1, unroll=8)
    def _transpose(g):
        rowsel = iota + g * 16
        jt = g // 8
        boff = (g % 8) * 16
        for j in range(EMBED_DIM):
            v = plsc.load_gather(rows_v, [rowsel, cols[j]])
            outbuf[j // 8, jt, j % 8, pl.ds(boff, 16)] = v

    jt0 = base // 128
    for ib in range(4):
        pltpu.sync_copy(outbuf.at[ib], out_hbm.at[ib, pl.ds(jt0, 4)])


def kernel(user_id, table):
    table33 = jnp.pad(table, ((0, 0), (0, 1)))
    out4 = _embed_gather(user_id, table33)
    return out4.transpose(1, 3, 0, 2).reshape(BATCH, EMBED_DIM)


# speedup vs baseline: 1.6351x; 1.0068x over previous
import functools

import jax
import jax.numpy as jnp
from jax import lax
from jax.experimental import pallas as pl
from jax.experimental.pallas import tpu as pltpu
from jax.experimental.pallas import tpu_sc as plsc

EMBED_DIM = 32
BATCH = 16384

NUM_CORES = 2
NUM_SUBCORES = 16
NUM_WORKERS = NUM_CORES * NUM_SUBCORES
B_PER_W = BATCH // NUM_WORKERS

_mesh = plsc.VectorSubcoreMesh(core_axis_name="c", subcore_axis_name="s")


@functools.partial(
    pl.kernel,
    mesh=_mesh,
    out_type=jax.ShapeDtypeStruct((4, 128, 8, 128), jnp.float32),
    scratch_types=[
        pltpu.VMEM((B_PER_W,), jnp.int32),
        pltpu.VMEM((B_PER_W, EMBED_DIM), jnp.float32),
        pltpu.VMEM((B_PER_W, EMBED_DIM + 1), jnp.float32),
        pltpu.VMEM((4, 4, 8, 128), jnp.float32),
        pltpu.SemaphoreType.DMA,
    ],
    compiler_params=pltpu.CompilerParams(
        use_tc_tiling_on_sc=False, needs_layout_passes=False
    ),
)
def _embed_gather(idx_hbm, table_hbm, out_hbm, idx_v, rows_v, rows_p, outbuf, sem):
    wid = lax.axis_index("s") * NUM_CORES + lax.axis_index("c")
    base = wid * B_PER_W
    pltpu.sync_copy(idx_hbm.at[pl.ds(base, B_PER_W)], idx_v)
    pltpu.async_copy(table_hbm.at[idx_v], rows_v, sem).wait()

    @plsc.parallel_loop(0, B_PER_W, step=1, unroll=8)
    def _repitch(r):
        rows_p[r, pl.ds(0, 16)] = rows_v[r, pl.ds(0, 16)]
        rows_p[r, pl.ds(16, 16)] = rows_v[r, pl.ds(16, 16)]

    iota = lax.iota(jnp.int32, 16)
    cols = [jnp.full((16,), j, jnp.int32) for j in range(EMBED_DIM)]

    @plsc.parallel_loop(0, B_PER_W // 16, step=1, unroll=8)
    def _transpose(g):
        rowsel = iota + g * 16
        jt = g // 8
        boff = (g % 8) * 16
        for j in range(EMBED_DIM):
            v = plsc.load_gather(rows_p, [rowsel, cols[j]])
            outbuf[j // 8, jt, j % 8, pl.ds(boff, 16)] = v

    jt0 = base // 128
    for ib in range(4):
        pltpu.sync_copy(outbuf.at[ib], out_hbm.at[ib, pl.ds(jt0, 4)])


def kernel(user_id, table):
    out4 = _embed_gather(user_id, table)
    return out4.transpose(1, 3, 0, 2).reshape(BATCH, EMBED_DIM)
